# baseline (speedup 1.0000x reference)
import functools

import jax
import jax.numpy as jnp
from jax import lax
from jax.experimental import pallas as pl
from jax.experimental.pallas import tpu as pltpu
from jax.experimental.pallas import tpu_sc as plsc

N = 10000
E = 320000
D = 128

NC = 2
NS = 16
NW = NC * NS

RPW = (-(-N // NW) + 7) // 8 * 8
NP = NW * RPW
C = 4000
NCHUNK = E // C
G = 128
LIST = C + G + 32

_NEG_INF = float("-inf")


def _segmax_sc(h, src, dst):
    mesh = plsc.VectorSubcoreMesh(core_axis_name="c", subcore_axis_name="s",
                                  num_cores=NC, num_subcores=NS)

    @functools.partial(
        pl.kernel,
        out_type=jax.ShapeDtypeStruct((NP, D), jnp.float32),
        mesh=mesh,
        compiler_params=pltpu.CompilerParams(needs_layout_passes=False),
        scratch_types=[
            pltpu.VMEM((RPW, D), jnp.float32),
            pltpu.VMEM((C,), jnp.int32),
            pltpu.VMEM((C,), jnp.int32),
            pltpu.VMEM((LIST,), jnp.int32),
            pltpu.VMEM((LIST,), jnp.int32),
            pltpu.VMEM((G, D), jnp.float32),
            pltpu.SemaphoreType.DMA,
        ],
    )
    def seg_kernel(h_hbm, src_hbm, dst_hbm, out_hbm,
                   acc, dstb, srcb, slist, dlist, rows, sem):
        wid = lax.axis_index("s") * NC + lax.axis_index("c")
        lo = wid * RPW
        hi = lo + RPW

        neg = jnp.full((16,), _NEG_INF, jnp.float32)

        @pl.loop(0, RPW)
        def _init(r):
            for k in range(D // 16):
                acc[r, pl.ds(k * 16, 16)] = neg

        zero16 = jnp.zeros((16,), jnp.int32)

        @pl.loop(0, NCHUNK)
        def _chunk(ci):
            base = ci * C
            pltpu.sync_copy(dst_hbm.at[pl.ds(base, C)], dstb)
            pltpu.sync_copy(src_hbm.at[pl.ds(base, C)], srcb)

            def scan_body(i, cnt):
                sl = pl.ds(i * 16, 16)
                dvec = dstb[sl]
                svec = srcb[sl]
                m = (dvec >= lo) & (dvec < hi)
                keys = jnp.where(m, dvec - lo, jnp.int32(0x7FFFFFFF))
                sk, sv = plsc.sort_key_val(keys, svec)
                slist[pl.ds(cnt, 16)] = sv
                dlist[pl.ds(cnt, 16)] = sk
                pc = plsc.all_reduce_population_count(m)
                return cnt + jnp.max(pc)

            cnt = lax.fori_loop(0, C // 16, scan_body, jnp.int32(0))

            nb = (cnt + (G - 1)) // G

            @pl.loop(cnt, nb * G, step=16)
            def _pad(j):
                slist[pl.ds(j, 16)] = zero16

            @pl.loop(0, nb)
            def _batch(b):
                pltpu.async_copy(h_hbm.at[slist.at[pl.ds(b * G, G)]],
                                 rows, sem).wait()
                jmax = jnp.minimum(jnp.int32(G), cnt - b * G)

                @pl.loop(0, jmax)
                def _apply(j):
                    d = dlist[pl.ds(b * G + j, 16)][0]
                    for k in range(D // 16):
                        sl = pl.ds(k * 16, 16)
                        acc[d, sl] = jnp.maximum(acc[d, sl], rows[j, sl])

        @pl.loop(0, RPW)
        def _fix(r):
            for k in range(D // 16):
                sl = pl.ds(k * 16, 16)
                v = acc[r, sl]
                acc[r, sl] = jnp.where(v == _NEG_INF, jnp.float32(0.0), v)

        pltpu.sync_copy(acc, out_hbm.at[pl.ds(lo, RPW)])

    return seg_kernel(h, src, dst)


def _linear_tc(agg, h, WlT, WrT, b2d, relu):
    BN = 2000
    grid = (N // BN,)

    def body(a_ref, h_ref, wl_ref, wr_ref, b_ref, o_ref):
        r = jnp.dot(a_ref[...], wl_ref[...],
                    preferred_element_type=jnp.float32)
        r = r + jnp.dot(h_ref[...], wr_ref[...],
                        preferred_element_type=jnp.float32)
        r = r + b_ref[...]
        if relu:
            r = jnp.maximum(r, 0.0)
        o_ref[...] = r

    return pl.pallas_call(
        body,
        grid=grid,
        in_specs=[
            pl.BlockSpec((BN, D), lambda i: (i, 0)),
            pl.BlockSpec((BN, D), lambda i: (i, 0)),
            pl.BlockSpec((D, D), lambda i: (0, 0)),
            pl.BlockSpec((D, D), lambda i: (0, 0)),
            pl.BlockSpec((1, D), lambda i: (0, 0)),
        ],
        out_specs=pl.BlockSpec((BN, D), lambda i: (i, 0)),
        out_shape=jax.ShapeDtypeStruct((N, D), jnp.float32),
    )(agg, h, WlT, WrT, b2d)


def kernel(x, edge_index, W1l, b1l, W1r, W2l, b2l, W2r):
    src = edge_index[0]
    dst = edge_index[1]

    agg1 = _segmax_sc(x, src, dst)[:N]
    h1 = _linear_tc(agg1, x, W1l.T, W1r.T, b1l.reshape(1, D), relu=True)
    agg2 = _segmax_sc(h1, src, dst)[:N]
    out = _linear_tc(agg2, h1, W2l.T, W2r.T, b2l.reshape(1, D), relu=False)
    return out.reshape(-1)
